# baseline (speedup 1.0000x reference)
import functools

import jax
import jax.numpy as jnp
from jax import lax
from jax.experimental import pallas as pl
from jax.experimental.pallas import tpu as pltpu
from jax.experimental.pallas import tpu_sc as plsc

N = 10000
E = 320000
D = 128
HD = D // 2
STEPS = 6

NC = 2
NS = 16
CHUNK = 256
NCHUNK = 1280
CPS = NCHUNK // NS
BLK = 20
NBLK = CPS // BLK
E_PAD = NCHUNK * CHUNK
NROW = 10112
RPS = NROW // NS
MROW = 624

NB = 2
LA = 1

RB = 400


def _edge_stage(m, srcs, dsts, zeros):
    mesh = plsc.VectorSubcoreMesh(core_axis_name="c", subcore_axis_name="s")

    @functools.partial(
        pl.kernel,
        out_type=jax.ShapeDtypeStruct((NROW, D), jnp.float32),
        mesh=mesh,
        compiler_params=pltpu.CompilerParams(use_tc_tiling_on_sc=False),
        scratch_types=[
            pltpu.VMEM((BLK, CHUNK), jnp.int32),
            pltpu.VMEM((BLK, CHUNK), jnp.int32),
            pltpu.VMEM((NB, CHUNK, HD), jnp.float32),
            pltpu.VMEM_SHARED((NROW, HD), jnp.float32),
            pltpu.VMEM_SHARED((NROW, HD), jnp.float32),
            pltpu.SemaphoreType.DMA((NB,)),
            pltpu.SemaphoreType.DMA((NB,)),
        ],
    )
    def k(m_hbm, src_hbm, dst_hbm, z_hbm, out_hbm,
          sidx_v, didx_v, rows_v, m_sh, acc_sh, sem_g, sem_s):
        cid = lax.axis_index("c")
        sid = lax.axis_index("s")

        pltpu.sync_copy(m_hbm.at[pl.ds(sid * MROW, MROW), pl.ds(cid * HD, HD)],
                        m_sh.at[pl.ds(sid * MROW, MROW)])

        @pl.when(sid == NS - 1)
        def _():
            pltpu.sync_copy(
                m_hbm.at[pl.ds(NS * MROW, N - NS * MROW), pl.ds(cid * HD, HD)],
                m_sh.at[pl.ds(NS * MROW, N - NS * MROW)])

        pltpu.sync_copy(z_hbm.at[pl.ds(sid * RPS, RPS)],
                        acc_sh.at[pl.ds(sid * RPS, RPS)])
        plsc.subcore_barrier()

        def fire_gather(j, b):
            pltpu.async_copy(m_sh.at[sidx_v.at[j]], rows_v.at[b], sem_g.at[b])

        def wait_gather(b):
            pltpu.make_async_copy(m_sh.at[sidx_v.at[0]], rows_v.at[b],
                                  sem_g.at[b]).wait()

        def fire_scatter(j, b):
            pltpu.async_copy(rows_v.at[b], acc_sh.at[didx_v.at[j]],
                             sem_s.at[b], add=True)

        def wait_scatter(b):
            pltpu.make_async_copy(rows_v.at[b], acc_sh.at[didx_v.at[0]],
                                  sem_s.at[b]).wait()

        for blk in range(NBLK):
            base = sid * CPS + blk * BLK
            pltpu.sync_copy(src_hbm.at[pl.ds(base, BLK)], sidx_v)
            pltpu.sync_copy(dst_hbm.at[pl.ds(base, BLK)], didx_v)

            for b in range(LA):
                fire_gather(b, b)

            @pl.loop(0, BLK, step=NB)
            def _(j0):
                for b in range(NB):
                    j = j0 + b
                    jn = j + LA
                    bn = (b + LA) % NB
                    @pl.when(jnp.logical_and(jn >= NB, jn < BLK))
                    def _():
                        wait_scatter(bn)

                    @pl.when(jn < BLK)
                    def _():
                        fire_gather(jn, bn)

                    wait_gather(b)
                    fire_scatter(j, b)

            for b in range(NB):
                wait_scatter(b)

        plsc.subcore_barrier()
        pltpu.sync_copy(acc_sh.at[pl.ds(sid * RPS, RPS)],
                        out_hbm.at[pl.ds(sid * RPS, RPS), pl.ds(cid * HD, HD)])

    return k(m, srcs, dsts, zeros)


def _mlp(h, W1, b1r, W2, b2r):
    def body(h_ref, w1_ref, b1_ref, w2_ref, b2_ref, o_ref):
        x = h_ref[...]
        t = lax.dot_general(x, w1_ref[...], (((1,), (1,)), ((), ())),
                            preferred_element_type=jnp.float32) + b1_ref[...]
        t = jnp.maximum(t, 0.0)
        o_ref[...] = lax.dot_general(t, w2_ref[...], (((1,), (1,)), ((), ())),
                                     preferred_element_type=jnp.float32) + b2_ref[...]

    return pl.pallas_call(
        body,
        grid=(N // RB,),
        in_specs=[
            pl.BlockSpec((RB, D), lambda i: (i, 0)),
            pl.BlockSpec((D, D), lambda i: (0, 0)),
            pl.BlockSpec((1, D), lambda i: (0, 0)),
            pl.BlockSpec((D, D), lambda i: (0, 0)),
            pl.BlockSpec((1, D), lambda i: (0, 0)),
        ],
        out_specs=pl.BlockSpec((RB, D), lambda i: (i, 0)),
        out_shape=jax.ShapeDtypeStruct((N, D), jnp.float32),
    )(h, W1, b1r, W2, b2r)


def _gh(h, W_hh, bhhr):
    def body(h_ref, whh_ref, bhh_ref, o_ref):
        o_ref[...] = lax.dot_general(h_ref[...], whh_ref[...],
                                     (((1,), (1,)), ((), ())),
                                     preferred_element_type=jnp.float32) + bhh_ref[...]

    return pl.pallas_call(
        body,
        grid=(N // RB,),
        in_specs=[
            pl.BlockSpec((RB, D), lambda i: (i, 0)),
            pl.BlockSpec((3 * D, D), lambda i: (0, 0)),
            pl.BlockSpec((1, 3 * D), lambda i: (0, 0)),
        ],
        out_specs=pl.BlockSpec((RB, 3 * D), lambda i: (i, 0)),
        out_shape=jax.ShapeDtypeStruct((N, 3 * D), jnp.float32),
    )(h, W_hh, bhhr)


def _fused(neigh_a, h, gh, W_ih, bihr, W1, b1r, W2, b2r):
    def body(p_ref, h_ref, gh_ref, wih_ref, bih_ref, w1_ref, b1_ref,
             w2_ref, b2_ref, oh_ref, om_ref):
        neigh = p_ref[...]
        hh = h_ref[...]
        gi = lax.dot_general(neigh, wih_ref[...], (((1,), (1,)), ((), ())),
                             preferred_element_type=jnp.float32) + bih_ref[...]
        ghv = gh_ref[...]
        r = jax.nn.sigmoid(gi[:, :D] + ghv[:, :D])
        z = jax.nn.sigmoid(gi[:, D:2 * D] + ghv[:, D:2 * D])
        n = jnp.tanh(gi[:, 2 * D:] + r * ghv[:, 2 * D:])
        h_new = (1.0 - z) * n + z * hh
        oh_ref[...] = h_new
        t = lax.dot_general(h_new, w1_ref[...], (((1,), (1,)), ((), ())),
                            preferred_element_type=jnp.float32) + b1_ref[...]
        t = jnp.maximum(t, 0.0)
        om_ref[...] = lax.dot_general(t, w2_ref[...], (((1,), (1,)), ((), ())),
                                      preferred_element_type=jnp.float32) + b2_ref[...]

    return pl.pallas_call(
        body,
        grid=(N // RB,),
        in_specs=[
            pl.BlockSpec((RB, D), lambda i: (i, 0)),
            pl.BlockSpec((RB, D), lambda i: (i, 0)),
            pl.BlockSpec((RB, 3 * D), lambda i: (i, 0)),
            pl.BlockSpec((3 * D, D), lambda i: (0, 0)),
            pl.BlockSpec((1, 3 * D), lambda i: (0, 0)),
            pl.BlockSpec((D, D), lambda i: (0, 0)),
            pl.BlockSpec((1, D), lambda i: (0, 0)),
            pl.BlockSpec((D, D), lambda i: (0, 0)),
            pl.BlockSpec((1, D), lambda i: (0, 0)),
        ],
        out_specs=[
            pl.BlockSpec((RB, D), lambda i: (i, 0)),
            pl.BlockSpec((RB, D), lambda i: (i, 0)),
        ],
        out_shape=[
            jax.ShapeDtypeStruct((N, D), jnp.float32),
            jax.ShapeDtypeStruct((N, D), jnp.float32),
        ],
    )(neigh_a, h, gh, W_ih, bihr, W1, b1r, W2, b2r)


def kernel(node_feats, edge_index, W1, b1, W2, b2, W_ih, W_hh, b_ih, b_hh):
    src = edge_index[0]
    dst = edge_index[1]
    pad = E_PAD - E
    srcs = jnp.concatenate([src, jnp.zeros((pad,), jnp.int32)]).reshape(NCHUNK, CHUNK)
    dsts = jnp.concatenate([dst, jnp.full((pad,), N, jnp.int32)]).reshape(NCHUNK, CHUNK)
    zeros = jnp.zeros((NROW, HD), jnp.float32)
    b1r = b1.reshape(1, D)
    b2r = b2.reshape(1, D)
    bihr = b_ih.reshape(1, 3 * D)
    bhhr = b_hh.reshape(1, 3 * D)

    h = node_feats
    m = _mlp(h, W1, b1r, W2, b2r)
    for _ in range(STEPS):
        gh = _gh(h, W_hh, bhhr)
        neigh = _edge_stage(m, srcs, dsts, zeros)
        h, m = _fused(neigh, h, gh, W_ih, bihr, W1, b1r, W2, b2r)
    return h


# speedup vs baseline: 1.2002x; 1.2002x over previous
import functools

import jax
import jax.numpy as jnp
from jax import lax
from jax.experimental import pallas as pl
from jax.experimental.pallas import tpu as pltpu
from jax.experimental.pallas import tpu_sc as plsc

N = 10000
E = 320000
D = 128
HD = D // 2
STEPS = 6

NC = 2
NS = 16
CHUNK = 128
NCHUNK = 2560
CPS = NCHUNK // NS
BLK = 40
NBLK = CPS // BLK
E_PAD = NCHUNK * CHUNK
NROW = 10112
RPS = NROW // NS
MROW = 624

NB = 4
LA = 2

RB = 400


def _edge_stage(m, srcs, dsts, zeros):
    mesh = plsc.VectorSubcoreMesh(core_axis_name="c", subcore_axis_name="s")

    @functools.partial(
        pl.kernel,
        out_type=jax.ShapeDtypeStruct((NROW, D), jnp.float32),
        mesh=mesh,
        compiler_params=pltpu.CompilerParams(use_tc_tiling_on_sc=False),
        scratch_types=[
            pltpu.VMEM((BLK, CHUNK), jnp.int32),
            pltpu.VMEM((BLK, CHUNK), jnp.int32),
            pltpu.VMEM((NB, CHUNK, HD), jnp.float32),
            pltpu.VMEM_SHARED((NROW, HD), jnp.float32),
            pltpu.VMEM_SHARED((NROW, HD), jnp.float32),
            pltpu.SemaphoreType.DMA((NB,)),
            pltpu.SemaphoreType.DMA((NB,)),
        ],
    )
    def k(m_hbm, src_hbm, dst_hbm, z_hbm, out_hbm,
          sidx_v, didx_v, rows_v, m_sh, acc_sh, sem_g, sem_s):
        cid = lax.axis_index("c")
        sid = lax.axis_index("s")

        pltpu.sync_copy(m_hbm.at[pl.ds(sid * MROW, MROW), pl.ds(cid * HD, HD)],
                        m_sh.at[pl.ds(sid * MROW, MROW)])

        @pl.when(sid == NS - 1)
        def _():
            pltpu.sync_copy(
                m_hbm.at[pl.ds(NS * MROW, N - NS * MROW), pl.ds(cid * HD, HD)],
                m_sh.at[pl.ds(NS * MROW, N - NS * MROW)])

        pltpu.sync_copy(z_hbm.at[pl.ds(sid * RPS, RPS)],
                        acc_sh.at[pl.ds(sid * RPS, RPS)])
        plsc.subcore_barrier()

        def fire_gather(j, b):
            pltpu.async_copy(m_sh.at[sidx_v.at[j]], rows_v.at[b], sem_g.at[b])

        def wait_gather(b):
            pltpu.make_async_copy(m_sh.at[sidx_v.at[0]], rows_v.at[b],
                                  sem_g.at[b]).wait()

        def fire_scatter(j, b):
            pltpu.async_copy(rows_v.at[b], acc_sh.at[didx_v.at[j]],
                             sem_s.at[b], add=True)

        def wait_scatter(b):
            pltpu.make_async_copy(rows_v.at[b], acc_sh.at[didx_v.at[0]],
                                  sem_s.at[b]).wait()

        for blk in range(NBLK):
            base = sid * CPS + blk * BLK
            pltpu.sync_copy(src_hbm.at[pl.ds(base, BLK)], sidx_v)
            pltpu.sync_copy(dst_hbm.at[pl.ds(base, BLK)], didx_v)

            for b in range(LA):
                fire_gather(b, b)

            @pl.loop(0, BLK, step=NB)
            def _(j0):
                for b in range(NB):
                    j = j0 + b
                    jn = j + LA
                    bn = (b + LA) % NB
                    @pl.when(jnp.logical_and(jn >= NB, jn < BLK))
                    def _():
                        wait_scatter(bn)

                    @pl.when(jn < BLK)
                    def _():
                        fire_gather(jn, bn)

                    wait_gather(b)
                    fire_scatter(j, b)

            for b in range(NB):
                wait_scatter(b)

        plsc.subcore_barrier()
        pltpu.sync_copy(acc_sh.at[pl.ds(sid * RPS, RPS)],
                        out_hbm.at[pl.ds(sid * RPS, RPS), pl.ds(cid * HD, HD)])

    return k(m, srcs, dsts, zeros)


def _mlp(h, W1, b1r, W2, b2r):
    def body(h_ref, w1_ref, b1_ref, w2_ref, b2_ref, o_ref):
        x = h_ref[...]
        t = lax.dot_general(x, w1_ref[...], (((1,), (1,)), ((), ())),
                            preferred_element_type=jnp.float32) + b1_ref[...]
        t = jnp.maximum(t, 0.0)
        o_ref[...] = lax.dot_general(t, w2_ref[...], (((1,), (1,)), ((), ())),
                                     preferred_element_type=jnp.float32) + b2_ref[...]

    return pl.pallas_call(
        body,
        grid=(N // RB,),
        in_specs=[
            pl.BlockSpec((RB, D), lambda i: (i, 0)),
            pl.BlockSpec((D, D), lambda i: (0, 0)),
            pl.BlockSpec((1, D), lambda i: (0, 0)),
            pl.BlockSpec((D, D), lambda i: (0, 0)),
            pl.BlockSpec((1, D), lambda i: (0, 0)),
        ],
        out_specs=pl.BlockSpec((RB, D), lambda i: (i, 0)),
        out_shape=jax.ShapeDtypeStruct((N, D), jnp.float32),
    )(h, W1, b1r, W2, b2r)


def _fused(neigh_a, h, W_ih, bihr, W_hh, bhhr, W1, b1r, W2, b2r):
    def body(p_ref, h_ref, wih_ref, bih_ref, whh_ref, bhh_ref, w1_ref, b1_ref,
             w2_ref, b2_ref, oh_ref, om_ref):
        neigh = p_ref[...]
        hh = h_ref[...]
        gi = lax.dot_general(neigh, wih_ref[...], (((1,), (1,)), ((), ())),
                             preferred_element_type=jnp.float32) + bih_ref[...]
        ghv = lax.dot_general(hh, whh_ref[...], (((1,), (1,)), ((), ())),
                              preferred_element_type=jnp.float32) + bhh_ref[...]
        r = jax.nn.sigmoid(gi[:, :D] + ghv[:, :D])
        z = jax.nn.sigmoid(gi[:, D:2 * D] + ghv[:, D:2 * D])
        n = jnp.tanh(gi[:, 2 * D:] + r * ghv[:, 2 * D:])
        h_new = (1.0 - z) * n + z * hh
        oh_ref[...] = h_new
        t = lax.dot_general(h_new, w1_ref[...], (((1,), (1,)), ((), ())),
                            preferred_element_type=jnp.float32) + b1_ref[...]
        t = jnp.maximum(t, 0.0)
        om_ref[...] = lax.dot_general(t, w2_ref[...], (((1,), (1,)), ((), ())),
                                      preferred_element_type=jnp.float32) + b2_ref[...]

    return pl.pallas_call(
        body,
        grid=(N // RB,),
        in_specs=[
            pl.BlockSpec((RB, D), lambda i: (i, 0)),
            pl.BlockSpec((RB, D), lambda i: (i, 0)),
            pl.BlockSpec((3 * D, D), lambda i: (0, 0)),
            pl.BlockSpec((1, 3 * D), lambda i: (0, 0)),
            pl.BlockSpec((3 * D, D), lambda i: (0, 0)),
            pl.BlockSpec((1, 3 * D), lambda i: (0, 0)),
            pl.BlockSpec((D, D), lambda i: (0, 0)),
            pl.BlockSpec((1, D), lambda i: (0, 0)),
            pl.BlockSpec((D, D), lambda i: (0, 0)),
            pl.BlockSpec((1, D), lambda i: (0, 0)),
        ],
        out_specs=[
            pl.BlockSpec((RB, D), lambda i: (i, 0)),
            pl.BlockSpec((RB, D), lambda i: (i, 0)),
        ],
        out_shape=[
            jax.ShapeDtypeStruct((N, D), jnp.float32),
            jax.ShapeDtypeStruct((N, D), jnp.float32),
        ],
    )(neigh_a, h, W_ih, bihr, W_hh, bhhr, W1, b1r, W2, b2r)


def kernel(node_feats, edge_index, W1, b1, W2, b2, W_ih, W_hh, b_ih, b_hh):
    src = edge_index[0]
    dst = edge_index[1]
    pad = E_PAD - E
    srcs = jnp.concatenate([src, jnp.zeros((pad,), jnp.int32)]).reshape(NCHUNK, CHUNK)
    dsts = jnp.concatenate([dst, jnp.full((pad,), N, jnp.int32)]).reshape(NCHUNK, CHUNK)
    zeros = jnp.zeros((NROW, HD), jnp.float32)
    b1r = b1.reshape(1, D)
    b2r = b2.reshape(1, D)
    bihr = b_ih.reshape(1, 3 * D)
    bhhr = b_hh.reshape(1, 3 * D)

    h = node_feats
    m = _mlp(h, W1, b1r, W2, b2r)
    for _ in range(STEPS):
        neigh = _edge_stage(m, srcs, dsts, zeros)
        h, m = _fused(neigh, h, W_ih, bihr, W_hh, bhhr, W1, b1r, W2, b2r)
    return h


# speedup vs baseline: 1.2049x; 1.0040x over previous
import functools

import jax
import jax.numpy as jnp
from jax import lax
from jax.experimental import pallas as pl
from jax.experimental.pallas import tpu as pltpu
from jax.experimental.pallas import tpu_sc as plsc

N = 10000
E = 320000
D = 128
HD = D // 2
STEPS = 6

NC = 2
NS = 16
CHUNK = 128
NCHUNK = 2560
CPS = NCHUNK // NS
BLK = 40
NBLK = CPS // BLK
E_PAD = NCHUNK * CHUNK
NROW = 10112
RPS = NROW // NS
MROW = 624

NB = 4
LA = 2

RB = 400


def _edge_stage(m, srcs, dsts, zeros):
    mesh = plsc.VectorSubcoreMesh(core_axis_name="c", subcore_axis_name="s")

    @functools.partial(
        pl.kernel,
        out_type=jax.ShapeDtypeStruct((NROW, D), jnp.float32),
        mesh=mesh,
        compiler_params=pltpu.CompilerParams(use_tc_tiling_on_sc=False),
        scratch_types=[
            pltpu.VMEM((BLK, CHUNK), jnp.int32),
            pltpu.VMEM((BLK, CHUNK), jnp.int32),
            pltpu.VMEM((NB, CHUNK, HD), jnp.float32),
            pltpu.VMEM_SHARED((NROW, HD), jnp.float32),
            pltpu.VMEM_SHARED((NROW, HD), jnp.float32),
            pltpu.SemaphoreType.DMA((NB,)),
            pltpu.SemaphoreType.DMA((NB,)),
        ],
    )
    def k(m_hbm, src_hbm, dst_hbm, z_hbm, out_hbm,
          sidx_v, didx_v, rows_v, m_sh, acc_sh, sem_g, sem_s):
        cid = lax.axis_index("c")
        sid = lax.axis_index("s")

        pltpu.sync_copy(m_hbm.at[pl.ds(sid * MROW, MROW), pl.ds(cid * HD, HD)],
                        m_sh.at[pl.ds(sid * MROW, MROW)])

        @pl.when(sid == NS - 1)
        def _():
            pltpu.sync_copy(
                m_hbm.at[pl.ds(NS * MROW, N - NS * MROW), pl.ds(cid * HD, HD)],
                m_sh.at[pl.ds(NS * MROW, N - NS * MROW)])

        pltpu.sync_copy(z_hbm.at[pl.ds(sid * RPS, RPS)],
                        acc_sh.at[pl.ds(sid * RPS, RPS)])
        plsc.subcore_barrier()

        def fire_gather(j, b):
            pltpu.async_copy(m_sh.at[sidx_v.at[j]], rows_v.at[b], sem_g.at[b])

        def wait_gather(b):
            pltpu.make_async_copy(m_sh.at[sidx_v.at[0]], rows_v.at[b],
                                  sem_g.at[b]).wait()

        def fire_scatter(j, b):
            pltpu.async_copy(rows_v.at[b], acc_sh.at[didx_v.at[j]],
                             sem_s.at[b], add=True)

        def wait_scatter(b):
            pltpu.make_async_copy(rows_v.at[b], acc_sh.at[didx_v.at[0]],
                                  sem_s.at[b]).wait()

        for blk in range(NBLK):
            base = sid * CPS + blk * BLK
            pltpu.sync_copy(src_hbm.at[pl.ds(base, BLK)], sidx_v)
            pltpu.sync_copy(dst_hbm.at[pl.ds(base, BLK)], didx_v)

            for b in range(LA):
                fire_gather(b, b)

            @pl.loop(0, BLK, step=NB)
            def _(j0):
                for b in range(NB):
                    j = j0 + b
                    jn = j + LA
                    bn = (b + LA) % NB
                    @pl.when(jnp.logical_and(jn >= NB, jn < BLK))
                    def _():
                        wait_scatter(bn)

                    @pl.when(jn < BLK)
                    def _():
                        fire_gather(jn, bn)

                    wait_gather(b)
                    fire_scatter(j, b)

            for b in range(NB):
                wait_scatter(b)

        plsc.subcore_barrier()
        pltpu.sync_copy(acc_sh.at[pl.ds(sid * RPS, RPS)],
                        out_hbm.at[pl.ds(sid * RPS, RPS), pl.ds(cid * HD, HD)])

    return k(m, srcs, dsts, zeros)


def _bf(x):
    return x.astype(jnp.bfloat16)


def _mlp(h, W1, b1r, W2, b2r):
    def body(h_ref, w1_ref, b1_ref, w2_ref, b2_ref, o_ref):
        x = h_ref[...]
        t = lax.dot_general(_bf(x), _bf(w1_ref[...]), (((1,), (1,)), ((), ())),
                            preferred_element_type=jnp.float32) + b1_ref[...]
        t = jnp.maximum(t, 0.0)
        o_ref[...] = lax.dot_general(_bf(t), _bf(w2_ref[...]),
                                     (((1,), (1,)), ((), ())),
                                     preferred_element_type=jnp.float32) + b2_ref[...]

    return pl.pallas_call(
        body,
        grid=(N // RB,),
        in_specs=[
            pl.BlockSpec((RB, D), lambda i: (i, 0)),
            pl.BlockSpec((D, D), lambda i: (0, 0)),
            pl.BlockSpec((1, D), lambda i: (0, 0)),
            pl.BlockSpec((D, D), lambda i: (0, 0)),
            pl.BlockSpec((1, D), lambda i: (0, 0)),
        ],
        out_specs=pl.BlockSpec((RB, D), lambda i: (i, 0)),
        out_shape=jax.ShapeDtypeStruct((N, D), jnp.float32),
    )(h, W1, b1r, W2, b2r)


def _fused(neigh_a, h, W_ih, bihr, W_hh, bhhr, W1, b1r, W2, b2r):
    def body(p_ref, h_ref, wih_ref, bih_ref, whh_ref, bhh_ref, w1_ref, b1_ref,
             w2_ref, b2_ref, oh_ref, om_ref):
        neigh = p_ref[...]
        hh = h_ref[...]
        gi = lax.dot_general(neigh, wih_ref[...], (((1,), (1,)), ((), ())),
                             preferred_element_type=jnp.float32) + bih_ref[...]
        ghv = lax.dot_general(hh, whh_ref[...], (((1,), (1,)), ((), ())),
                              preferred_element_type=jnp.float32) + bhh_ref[...]
        r = jax.nn.sigmoid(gi[:, :D] + ghv[:, :D])
        z = jax.nn.sigmoid(gi[:, D:2 * D] + ghv[:, D:2 * D])
        n = jnp.tanh(gi[:, 2 * D:] + r * ghv[:, 2 * D:])
        h_new = (1.0 - z) * n + z * hh
        oh_ref[...] = h_new
        t = lax.dot_general(_bf(h_new), _bf(w1_ref[...]),
                            (((1,), (1,)), ((), ())),
                            preferred_element_type=jnp.float32) + b1_ref[...]
        t = jnp.maximum(t, 0.0)
        om_ref[...] = lax.dot_general(_bf(t), _bf(w2_ref[...]),
                                      (((1,), (1,)), ((), ())),
                                      preferred_element_type=jnp.float32) + b2_ref[...]

    return pl.pallas_call(
        body,
        grid=(N // RB,),
        in_specs=[
            pl.BlockSpec((RB, D), lambda i: (i, 0)),
            pl.BlockSpec((RB, D), lambda i: (i, 0)),
            pl.BlockSpec((3 * D, D), lambda i: (0, 0)),
            pl.BlockSpec((1, 3 * D), lambda i: (0, 0)),
            pl.BlockSpec((3 * D, D), lambda i: (0, 0)),
            pl.BlockSpec((1, 3 * D), lambda i: (0, 0)),
            pl.BlockSpec((D, D), lambda i: (0, 0)),
            pl.BlockSpec((1, D), lambda i: (0, 0)),
            pl.BlockSpec((D, D), lambda i: (0, 0)),
            pl.BlockSpec((1, D), lambda i: (0, 0)),
        ],
        out_specs=[
            pl.BlockSpec((RB, D), lambda i: (i, 0)),
            pl.BlockSpec((RB, D), lambda i: (i, 0)),
        ],
        out_shape=[
            jax.ShapeDtypeStruct((N, D), jnp.float32),
            jax.ShapeDtypeStruct((N, D), jnp.float32),
        ],
    )(neigh_a, h, W_ih, bihr, W_hh, bhhr, W1, b1r, W2, b2r)


def kernel(node_feats, edge_index, W1, b1, W2, b2, W_ih, W_hh, b_ih, b_hh):
    src = edge_index[0]
    dst = edge_index[1]
    pad = E_PAD - E
    srcs = jnp.concatenate([src, jnp.zeros((pad,), jnp.int32)]).reshape(NCHUNK, CHUNK)
    dsts = jnp.concatenate([dst, jnp.full((pad,), N, jnp.int32)]).reshape(NCHUNK, CHUNK)
    zeros = jnp.zeros((NROW, HD), jnp.float32)
    b1r = b1.reshape(1, D)
    b2r = b2.reshape(1, D)
    bihr = b_ih.reshape(1, 3 * D)
    bhhr = b_hh.reshape(1, 3 * D)

    h = node_feats
    m = _mlp(h, W1, b1r, W2, b2r)
    for _ in range(STEPS):
        neigh = _edge_stage(m, srcs, dsts, zeros)
        h, m = _fused(neigh, h, W_ih, bihr, W_hh, bhhr, W1, b1r, W2, b2r)
    return h
